# Initial kernel scaffold; baseline (speedup 1.0000x reference)
import functools

import jax
import jax.numpy as jnp
from jax import lax
from jax.experimental import pallas as pl
from jax.experimental.pallas import tpu as pltpu
from jax.experimental.pallas import tpu_sc as plsc

N_NODES = 10000
N_EDGES = 320000
D_FEAT = 128
CONV_H = 128
LIN_H = 64
NUM_CLASSES = 10
NUM_GRAPHS = 64

_NC = 2
_NS = 16
_NW = _NC * _NS
_EPT = N_EDGES // _NW
_CH = 128
_NCHUNK = N_EDGES // _CH
_BASE = _NCHUNK // _NW
_XTRA = _NCHUNK % _NW
_RPT = N_NODES // _NS


def _deg_body(col_hbm, ew_hbm, out_hbm, col_v, w_v, deg_v):
    cid = lax.axis_index("c")
    sid = lax.axis_index("s")
    wid = sid * _NC + cid
    base = wid * _EPT
    pltpu.sync_copy(col_hbm.at[pl.ds(base, _EPT)], col_v)
    pltpu.sync_copy(ew_hbm.at[pl.ds(base, _EPT)], w_v)

    def zero(i, _):
        deg_v[pl.ds(i * 16, 16)] = jnp.zeros((16,), jnp.float32)
        return 0

    lax.fori_loop(0, N_NODES // 16, zero, 0)

    def acc(i, _):
        sl = pl.ds(i * 16, 16)
        plsc.addupdate_scatter(deg_v, [col_v[sl]], w_v[sl])
        return 0

    lax.fori_loop(0, _EPT // 16, acc, 0)
    pltpu.sync_copy(deg_v, out_hbm.at[wid])


def _deg_partials(col, ew):
    mesh = plsc.VectorSubcoreMesh(core_axis_name="c", subcore_axis_name="s")
    return pl.kernel(
        _deg_body,
        out_type=jax.ShapeDtypeStruct((_NW, N_NODES), jnp.float32),
        mesh=mesh,
        scratch_types=[
            pltpu.VMEM((_EPT,), jnp.int32),
            pltpu.VMEM((_EPT,), jnp.float32),
            pltpu.VMEM((N_NODES,), jnp.float32),
        ],
    )(col, ew)


def _tc1_body(x_ref, w_ref, degp_ref, h_ref, dinv_ref):
    h_ref[...] = jnp.dot(x_ref[...], w_ref[...],
                         preferred_element_type=jnp.float32)
    deg = jnp.sum(degp_ref[...], axis=0, keepdims=True) + 1.0
    dinv_ref[...] = lax.rsqrt(deg)


def _tc1(x, w1t, degp):
    nb = 10
    blk = N_NODES // nb
    return pl.pallas_call(
        _tc1_body,
        grid=(nb,),
        in_specs=[
            pl.BlockSpec((blk, D_FEAT), lambda i: (i, 0)),
            pl.BlockSpec((D_FEAT, CONV_H), lambda i: (0, 0)),
            pl.BlockSpec((_NW, N_NODES), lambda i: (0, 0)),
        ],
        out_specs=[
            pl.BlockSpec((blk, CONV_H), lambda i: (i, 0)),
            pl.BlockSpec((1, N_NODES), lambda i: (0, 0)),
        ],
        out_shape=[
            jax.ShapeDtypeStruct((N_NODES, CONV_H), jnp.float32),
            jax.ShapeDtypeStruct((1, N_NODES), jnp.float32),
        ],
    )(x, w1t, degp)


def _msg_body(h_hbm, row_hbm, col_hbm, ew_hbm, dinv_hbm, zero_hbm, part_hbm,
              dinv_v, ridx_v, cidx_v, wch_v, nrm_v, rows_v, accum, gsem):
    cid = lax.axis_index("c")
    sid = lax.axis_index("s")
    wid = sid * _NC + cid

    pltpu.sync_copy(dinv_hbm, dinv_v)

    @pl.when(sid == 0)
    def _():
        pltpu.sync_copy(zero_hbm, accum)

    plsc.subcore_barrier()

    nchunks = jnp.where(wid < _XTRA, _BASE + 1, _BASE)
    start = _BASE * wid + jnp.minimum(wid, _XTRA)

    def body(j, _):
        base = (start + j) * _CH
        pltpu.sync_copy(row_hbm.at[pl.ds(base, _CH)], ridx_v)
        pltpu.sync_copy(col_hbm.at[pl.ds(base, _CH)], cidx_v)
        pltpu.sync_copy(ew_hbm.at[pl.ds(base, _CH)], wch_v)
        pltpu.async_copy(h_hbm.at[ridx_v], rows_v, gsem).wait()
        for k in range(_CH // 16):
            sl = pl.ds(k * 16, 16)
            dr = plsc.load_gather(dinv_v, [ridx_v[sl]])
            dc = plsc.load_gather(dinv_v, [cidx_v[sl]])
            nrm_v[sl] = dr * wch_v[sl] * dc

        def scale(b, _):
            s = nrm_v[b]
            for f in range(D_FEAT // 16):
                fs = pl.ds(f * 16, 16)
                rows_v[b, fs] = rows_v[b, fs] * s
            return 0

        lax.fori_loop(0, _CH, scale, 0)
        pltpu.sync_copy(rows_v, accum.at[cidx_v], add=True)
        return 0

    lax.fori_loop(0, nchunks, body, 0)
    plsc.subcore_barrier()

    rb = sid * _RPT
    pltpu.sync_copy(accum.at[pl.ds(rb, _RPT)],
                    part_hbm.at[cid].at[pl.ds(rb, _RPT)])


def _msg_partials(h, row, col, ew, dinv, zeros):
    mesh = plsc.VectorSubcoreMesh(core_axis_name="c", subcore_axis_name="s")
    return pl.kernel(
        _msg_body,
        out_type=jax.ShapeDtypeStruct((_NC, N_NODES, CONV_H), jnp.float32),
        mesh=mesh,
        scratch_types=[
            pltpu.VMEM((N_NODES,), jnp.float32),
            pltpu.VMEM((_CH,), jnp.int32),
            pltpu.VMEM((_CH,), jnp.int32),
            pltpu.VMEM((_CH,), jnp.float32),
            pltpu.VMEM((_CH,), jnp.float32),
            pltpu.VMEM((_CH, CONV_H), jnp.float32),
            pltpu.VMEM_SHARED((N_NODES, CONV_H), jnp.float32),
            pltpu.SemaphoreType.DMA,
        ],
    )(h, row, col, ew, dinv, zeros)


def _tc2_body(part_ref, h_ref, dinv_ref, b1_ref, batch_ref,
              l1wt_ref, l1b_ref, l2wt_ref, l2b_ref, out_ref, pool_acc):
    i = pl.program_id(0)
    d = dinv_ref[...]
    o = part_ref[0] + part_ref[1] + h_ref[...] * (d * d) + b1_ref[...]
    o = jnp.maximum(o, 0.0)
    bid = batch_ref[...]

    @pl.when(i == 0)
    def _():
        pool_acc[...] = jnp.full((NUM_GRAPHS, CONV_H), -jnp.inf,
                                 dtype=jnp.float32)

    for g in range(NUM_GRAPHS):
        og = jnp.where(bid == g, o, -jnp.inf)
        cg = jnp.max(og, axis=0, keepdims=True)
        sl = pl.ds(g, 1)
        pool_acc[sl, :] = jnp.maximum(pool_acc[sl, :], cg)

    @pl.when(i == pl.num_programs(0) - 1)
    def _():
        p = pool_acc[...]
        z = jnp.dot(p, l1wt_ref[...], preferred_element_type=jnp.float32)
        z = jnp.maximum(z + l1b_ref[...], 0.0)
        out_ref[...] = (jnp.dot(z, l2wt_ref[...],
                                preferred_element_type=jnp.float32)
                        + l2b_ref[...])


def _tc2(part, h, dinv_col, b1, batch_col, l1wt, l1b, l2wt, l2b):
    nb = 10
    blk = N_NODES // nb
    return pl.pallas_call(
        _tc2_body,
        grid=(nb,),
        in_specs=[
            pl.BlockSpec((_NC, blk, CONV_H), lambda i: (0, i, 0)),
            pl.BlockSpec((blk, CONV_H), lambda i: (i, 0)),
            pl.BlockSpec((blk, 1), lambda i: (i, 0)),
            pl.BlockSpec((1, CONV_H), lambda i: (0, 0)),
            pl.BlockSpec((blk, 1), lambda i: (i, 0)),
            pl.BlockSpec((CONV_H, LIN_H), lambda i: (0, 0)),
            pl.BlockSpec((1, LIN_H), lambda i: (0, 0)),
            pl.BlockSpec((LIN_H, NUM_CLASSES), lambda i: (0, 0)),
            pl.BlockSpec((1, NUM_CLASSES), lambda i: (0, 0)),
        ],
        out_specs=pl.BlockSpec((NUM_GRAPHS, NUM_CLASSES), lambda i: (0, 0)),
        out_shape=jax.ShapeDtypeStruct((NUM_GRAPHS, NUM_CLASSES), jnp.float32),
        scratch_shapes=[pltpu.VMEM((NUM_GRAPHS, CONV_H), jnp.float32)],
    )(part, h, dinv_col, b1, batch_col, l1wt, l1b, l2wt, l2b)


def kernel(x, edge_index, edge_weight, batch, W1, b1, lin1_w, lin1_b,
           lin2_w, lin2_b):
    row = edge_index[0]
    col = edge_index[1]
    ew = jnp.ravel(edge_weight).astype(jnp.float32)
    x = x.astype(jnp.float32)

    degp = _deg_partials(col, ew)
    h, dinv_row = _tc1(x, W1.T, degp)
    dinv = dinv_row.reshape(N_NODES)
    zeros = jnp.zeros((N_NODES, CONV_H), jnp.float32)
    part = _msg_partials(h, row, col, ew, dinv, zeros)
    out = _tc2(part, h, dinv.reshape(N_NODES, 1), b1.reshape(1, CONV_H),
               batch.reshape(N_NODES, 1).astype(jnp.int32),
               lin1_w.T, lin1_b.reshape(1, LIN_H),
               lin2_w.T, lin2_b.reshape(1, NUM_CLASSES))
    return out


# speedup vs baseline: 14.4046x; 14.4046x over previous
import functools

import jax
import jax.numpy as jnp
from jax import lax
from jax.experimental import pallas as pl
from jax.experimental.pallas import tpu as pltpu
from jax.experimental.pallas import tpu_sc as plsc

N_NODES = 10000
N_EDGES = 320000
D_FEAT = 128
CONV_H = 128
LIN_H = 64
NUM_CLASSES = 10
NUM_GRAPHS = 64

_NC = 2
_NS = 16
_NW = _NC * _NS
_EPT = N_EDGES // _NW
_CH = 128
_NCHUNK = N_EDGES // _CH
_BASE = _NCHUNK // _NW
_XTRA = _NCHUNK % _NW
_RPT = N_NODES // _NS


def _deg_body(col_hbm, ew_hbm, out_hbm, col_v, w_v, deg_v):
    cid = lax.axis_index("c")
    sid = lax.axis_index("s")
    wid = sid * _NC + cid
    base = wid * _EPT
    pltpu.sync_copy(col_hbm.at[pl.ds(base, _EPT)], col_v)
    pltpu.sync_copy(ew_hbm.at[pl.ds(base, _EPT)], w_v)

    def zero(i, _):
        deg_v[pl.ds(i * 16, 16)] = jnp.zeros((16,), jnp.float32)
        return 0

    lax.fori_loop(0, N_NODES // 16, zero, 0)

    def acc(i, _):
        sl = pl.ds(i * 16, 16)
        plsc.addupdate_scatter(deg_v, [col_v[sl]], w_v[sl])
        return 0

    lax.fori_loop(0, _EPT // 16, acc, 0)
    pltpu.sync_copy(deg_v, out_hbm.at[wid])


def _deg_partials(col, ew):
    mesh = plsc.VectorSubcoreMesh(core_axis_name="c", subcore_axis_name="s")
    return pl.kernel(
        _deg_body,
        out_type=jax.ShapeDtypeStruct((_NW, N_NODES), jnp.float32),
        mesh=mesh,
        compiler_params=pltpu.CompilerParams(needs_layout_passes=False),
        scratch_types=[
            pltpu.VMEM((_EPT,), jnp.int32),
            pltpu.VMEM((_EPT,), jnp.float32),
            pltpu.VMEM((N_NODES,), jnp.float32),
        ],
    )(col, ew)


def _tc1_body(x_ref, w_ref, degp_ref, h_ref, dinv_ref):
    h_ref[...] = jnp.dot(x_ref[...], w_ref[...],
                         preferred_element_type=jnp.float32)
    deg = jnp.sum(degp_ref[...], axis=0, keepdims=True) + 1.0
    dinv_ref[...] = lax.rsqrt(deg)


def _tc1(x, w1t, degp):
    nb = 10
    blk = N_NODES // nb
    return pl.pallas_call(
        _tc1_body,
        grid=(nb,),
        in_specs=[
            pl.BlockSpec((blk, D_FEAT), lambda i: (i, 0)),
            pl.BlockSpec((D_FEAT, CONV_H), lambda i: (0, 0)),
            pl.BlockSpec((_NW, N_NODES), lambda i: (0, 0)),
        ],
        out_specs=[
            pl.BlockSpec((blk, CONV_H), lambda i: (i, 0)),
            pl.BlockSpec((1, N_NODES), lambda i: (0, 0)),
        ],
        out_shape=[
            jax.ShapeDtypeStruct((N_NODES, CONV_H), jnp.float32),
            jax.ShapeDtypeStruct((1, N_NODES), jnp.float32),
        ],
    )(x, w1t, degp)


def _msg_body(h_hbm, row_hbm, col_hbm, ew_hbm, dinv_hbm, zero_hbm, part_hbm,
              dinv_v, ridx_v, cidx_v, wch_v, nrm_v, rows_v, accum, gsem):
    cid = lax.axis_index("c")
    sid = lax.axis_index("s")
    wid = sid * _NC + cid

    pltpu.sync_copy(dinv_hbm, dinv_v)

    @pl.when(sid == 0)
    def _():
        pltpu.sync_copy(zero_hbm, accum)

    plsc.subcore_barrier()

    nchunks = jnp.where(wid < _XTRA, _BASE + 1, _BASE)
    start = _BASE * wid + jnp.minimum(wid, _XTRA)

    def body(j, _):
        base = (start + j) * _CH
        pltpu.sync_copy(row_hbm.at[pl.ds(base, _CH)], ridx_v)
        pltpu.sync_copy(col_hbm.at[pl.ds(base, _CH)], cidx_v)
        pltpu.sync_copy(ew_hbm.at[pl.ds(base, _CH)], wch_v)
        pltpu.async_copy(h_hbm.at[ridx_v], rows_v, gsem).wait()
        for k in range(_CH // 16):
            sl = pl.ds(k * 16, 16)
            dr = plsc.load_gather(dinv_v, [ridx_v[sl]])
            dc = plsc.load_gather(dinv_v, [cidx_v[sl]])
            nrm_v[sl] = dr * wch_v[sl] * dc

        def scale(b, _):
            s = plsc.load_gather(nrm_v, [jnp.full((16,), b, jnp.int32)])
            for f in range(D_FEAT // 16):
                fs = pl.ds(f * 16, 16)
                rows_v[b, fs] = rows_v[b, fs] * s
            return 0

        lax.fori_loop(0, _CH, scale, 0)
        pltpu.sync_copy(rows_v, accum.at[cidx_v], add=True)
        return 0

    lax.fori_loop(0, nchunks, body, 0)
    plsc.subcore_barrier()

    rb = sid * _RPT
    pltpu.sync_copy(accum.at[pl.ds(rb, _RPT)],
                    part_hbm.at[cid].at[pl.ds(rb, _RPT)])


def _msg_partials(h, row, col, ew, dinv, zeros):
    mesh = plsc.VectorSubcoreMesh(core_axis_name="c", subcore_axis_name="s")
    return pl.kernel(
        _msg_body,
        out_type=jax.ShapeDtypeStruct((_NC, N_NODES, CONV_H), jnp.float32),
        mesh=mesh,
        compiler_params=pltpu.CompilerParams(needs_layout_passes=False,
                                             use_tc_tiling_on_sc=False),
        scratch_types=[
            pltpu.VMEM((N_NODES,), jnp.float32),
            pltpu.VMEM((_CH,), jnp.int32),
            pltpu.VMEM((_CH,), jnp.int32),
            pltpu.VMEM((_CH,), jnp.float32),
            pltpu.VMEM((_CH,), jnp.float32),
            pltpu.VMEM((_CH, CONV_H), jnp.float32),
            pltpu.VMEM_SHARED((N_NODES, CONV_H), jnp.float32),
            pltpu.SemaphoreType.DMA,
        ],
    )(h, row, col, ew, dinv, zeros)


def _tc2_body(part_ref, h_ref, dinv_ref, b1_ref, batch_ref,
              l1wt_ref, l1b_ref, l2wt_ref, l2b_ref, out_ref, pool_acc):
    i = pl.program_id(0)
    d = dinv_ref[...]
    o = part_ref[0] + part_ref[1] + h_ref[...] * (d * d) + b1_ref[...]
    o = jnp.maximum(o, 0.0)
    bid = batch_ref[...]

    @pl.when(i == 0)
    def _():
        pool_acc[...] = jnp.full((NUM_GRAPHS, CONV_H), -jnp.inf,
                                 dtype=jnp.float32)

    for g in range(NUM_GRAPHS):
        og = jnp.where(bid == g, o, -jnp.inf)
        cg = jnp.max(og, axis=0, keepdims=True)
        sl = pl.ds(g, 1)
        pool_acc[sl, :] = jnp.maximum(pool_acc[sl, :], cg)

    @pl.when(i == pl.num_programs(0) - 1)
    def _():
        p = pool_acc[...]
        z = jnp.dot(p, l1wt_ref[...], preferred_element_type=jnp.float32)
        z = jnp.maximum(z + l1b_ref[...], 0.0)
        out_ref[...] = (jnp.dot(z, l2wt_ref[...],
                                preferred_element_type=jnp.float32)
                        + l2b_ref[...])


def _tc2(part, h, dinv_col, b1, batch_col, l1wt, l1b, l2wt, l2b):
    nb = 10
    blk = N_NODES // nb
    return pl.pallas_call(
        _tc2_body,
        grid=(nb,),
        in_specs=[
            pl.BlockSpec((_NC, blk, CONV_H), lambda i: (0, i, 0)),
            pl.BlockSpec((blk, CONV_H), lambda i: (i, 0)),
            pl.BlockSpec((blk, 1), lambda i: (i, 0)),
            pl.BlockSpec((1, CONV_H), lambda i: (0, 0)),
            pl.BlockSpec((blk, 1), lambda i: (i, 0)),
            pl.BlockSpec((CONV_H, LIN_H), lambda i: (0, 0)),
            pl.BlockSpec((1, LIN_H), lambda i: (0, 0)),
            pl.BlockSpec((LIN_H, NUM_CLASSES), lambda i: (0, 0)),
            pl.BlockSpec((1, NUM_CLASSES), lambda i: (0, 0)),
        ],
        out_specs=pl.BlockSpec((NUM_GRAPHS, NUM_CLASSES), lambda i: (0, 0)),
        out_shape=jax.ShapeDtypeStruct((NUM_GRAPHS, NUM_CLASSES), jnp.float32),
        scratch_shapes=[pltpu.VMEM((NUM_GRAPHS, CONV_H), jnp.float32)],
    )(part, h, dinv_col, b1, batch_col, l1wt, l1b, l2wt, l2b)


def kernel(x, edge_index, edge_weight, batch, W1, b1, lin1_w, lin1_b,
           lin2_w, lin2_b):
    row = edge_index[0]
    col = edge_index[1]
    ew = jnp.ravel(edge_weight).astype(jnp.float32)
    x = x.astype(jnp.float32)

    degp = _deg_partials(col, ew)
    h, dinv_row = _tc1(x, W1.T, degp)
    dinv = dinv_row.reshape(N_NODES)
    zeros = jnp.zeros((N_NODES, CONV_H), jnp.float32)
    part = _msg_partials(h, row, col, ew, dinv, zeros)
    out = _tc2(part, h, dinv.reshape(N_NODES, 1), b1.reshape(1, CONV_H),
               batch.reshape(N_NODES, 1).astype(jnp.int32),
               lin1_w.T, lin1_b.reshape(1, LIN_H),
               lin2_w.T, lin2_b.reshape(1, NUM_CLASSES))
    return out


# speedup vs baseline: 21.3439x; 1.4817x over previous
import functools

import jax
import jax.numpy as jnp
from jax import lax
from jax.experimental import pallas as pl
from jax.experimental.pallas import tpu as pltpu
from jax.experimental.pallas import tpu_sc as plsc

N_NODES = 10000
N_EDGES = 320000
D_FEAT = 128
CONV_H = 128
LIN_H = 64
NUM_CLASSES = 10
NUM_GRAPHS = 64

_NC = 2
_NS = 16
_NW = _NC * _NS
_EPT = N_EDGES // _NW
_CH = 80
_NCHUNK = N_EDGES // _CH
_CPT = _NCHUNK // _NW
_NBUF = 3
_MAIN = _CPT - _CPT % _NBUF
_GRP = _CH // 16
_RPT = N_NODES // _NS


def _deg_body(col_hbm, ew_hbm, out_hbm, col_v, w_v, deg_v):
    cid = lax.axis_index("c")
    sid = lax.axis_index("s")
    wid = sid * _NC + cid
    base = wid * _EPT
    pltpu.sync_copy(col_hbm.at[pl.ds(base, _EPT)], col_v)
    pltpu.sync_copy(ew_hbm.at[pl.ds(base, _EPT)], w_v)

    def zero(i, _):
        deg_v[pl.ds(i * 16, 16)] = jnp.zeros((16,), jnp.float32)
        return 0

    lax.fori_loop(0, N_NODES // 16, zero, 0)

    def acc(i, _):
        sl = pl.ds(i * 16, 16)
        plsc.addupdate_scatter(deg_v, [col_v[sl]], w_v[sl])
        return 0

    lax.fori_loop(0, _EPT // 16, acc, 0)
    pltpu.sync_copy(deg_v, out_hbm.at[wid])


def _deg_partials(col, ew):
    mesh = plsc.VectorSubcoreMesh(core_axis_name="c", subcore_axis_name="s")
    return pl.kernel(
        _deg_body,
        out_type=jax.ShapeDtypeStruct((_NW, N_NODES), jnp.float32),
        mesh=mesh,
        compiler_params=pltpu.CompilerParams(needs_layout_passes=False),
        scratch_types=[
            pltpu.VMEM((_EPT,), jnp.int32),
            pltpu.VMEM((_EPT,), jnp.float32),
            pltpu.VMEM((N_NODES,), jnp.float32),
        ],
    )(col, ew)


def _tc1_body(x_ref, w_ref, degp_ref, h_ref, dinv_ref):
    h_ref[...] = jnp.dot(x_ref[...], w_ref[...],
                         preferred_element_type=jnp.float32)
    deg = jnp.sum(degp_ref[...], axis=0, keepdims=True) + 1.0
    dinv_ref[...] = lax.rsqrt(deg)


def _tc1(x, w1t, degp):
    nb = 10
    blk = N_NODES // nb
    return pl.pallas_call(
        _tc1_body,
        grid=(nb,),
        in_specs=[
            pl.BlockSpec((blk, D_FEAT), lambda i: (i, 0)),
            pl.BlockSpec((D_FEAT, CONV_H), lambda i: (0, 0)),
            pl.BlockSpec((_NW, N_NODES), lambda i: (0, 0)),
        ],
        out_specs=[
            pl.BlockSpec((blk, CONV_H), lambda i: (i, 0)),
            pl.BlockSpec((1, N_NODES), lambda i: (0, 0)),
        ],
        out_shape=[
            jax.ShapeDtypeStruct((N_NODES, CONV_H), jnp.float32),
            jax.ShapeDtypeStruct((1, N_NODES), jnp.float32),
        ],
    )(x, w1t, degp)


def _msg_body(h_hbm, row_hbm, col_hbm, ew_hbm, dinv_hbm, zero_hbm, part_hbm,
              dinv_v, nrm_v, *rest):
    n = _NBUF
    ridx = rest[0:n]
    cidx = rest[n:2 * n]
    wch = rest[2 * n:3 * n]
    rows = rest[3 * n:4 * n]
    esem = rest[4 * n:5 * n]
    gsem = rest[5 * n:6 * n]
    ssem = rest[6 * n:7 * n]
    accum = rest[7 * n]

    cid = lax.axis_index("c")
    sid = lax.axis_index("s")
    wid = sid * _NC + cid
    cbase = wid * _CPT

    pltpu.sync_copy(dinv_hbm, dinv_v)

    @pl.when(sid == 0)
    def _():
        pltpu.sync_copy(zero_hbm, accum)

    plsc.subcore_barrier()

    def stage(c, s):
        pltpu.async_copy(row_hbm.at[c], ridx[s], esem[s])
        pltpu.async_copy(col_hbm.at[c], cidx[s], esem[s])
        pltpu.async_copy(ew_hbm.at[c], wch[s], esem[s])

    def ewait(s):
        pltpu.make_async_copy(row_hbm.at[0], ridx[s], esem[s]).wait()
        pltpu.make_async_copy(col_hbm.at[0], cidx[s], esem[s]).wait()
        pltpu.make_async_copy(ew_hbm.at[0], wch[s], esem[s]).wait()

    def gstart(s):
        pltpu.async_copy(h_hbm.at[ridx[s]], rows[s], gsem[s])

    def gwait(s):
        pltpu.make_async_copy(h_hbm.at[pl.ds(0, _CH)], rows[s],
                              gsem[s]).wait()

    def swait(s):
        pltpu.make_async_copy(rows[s], accum.at[pl.ds(0, _CH)],
                              ssem[s]).wait()

    def compute_scatter(s):
        for k in range(_GRP):
            sl = pl.ds(k * 16, 16)
            dr = plsc.load_gather(dinv_v, [ridx[s][sl]])
            dc = plsc.load_gather(dinv_v, [cidx[s][sl]])
            nrm_v[sl] = dr * wch[s][sl] * dc

        def scale(g, _):
            nrm16 = nrm_v[pl.ds(g * 16, 16)]
            for i16 in range(16):
                ei = g * 16 + i16
                sc = nrm16[i16]
                for f in range(D_FEAT // 16):
                    fs = pl.ds(f * 16, 16)
                    rows[s][ei, fs] = rows[s][ei, fs] * sc
            return 0

        lax.fori_loop(0, _GRP, scale, 0)
        pltpu.async_copy(rows[s], accum.at[cidx[s]], ssem[s], add=True)

    stage(cbase, 0)
    stage(cbase + 1, 1)
    ewait(0)
    gstart(0)

    def outer(t, _):
        for b in range(n):
            j = t * n + b
            gwait(b)
            compute_scatter(b)
            s2 = (b + 2) % n

            @pl.when(j + 2 < _MAIN)
            def _():
                @pl.when(j >= 1)
                def _():
                    swait(s2)

                stage(cbase + j + 2, s2)

            s1 = (b + 1) % n

            @pl.when(j + 1 < _MAIN)
            def _():
                ewait(s1)
                gstart(s1)

        return 0

    lax.fori_loop(0, _MAIN // n, outer, 0)

    for b in range(n):
        swait(b)

    for jx in range(_MAIN, _CPT):
        stage(cbase + jx, 0)
        ewait(0)
        gstart(0)
        gwait(0)
        compute_scatter(0)
        swait(0)

    plsc.subcore_barrier()

    rb = sid * _RPT
    pltpu.sync_copy(accum.at[pl.ds(rb, _RPT)],
                    part_hbm.at[cid].at[pl.ds(rb, _RPT)])


def _msg_partials(h, row, col, ew, dinv, zeros):
    mesh = plsc.VectorSubcoreMesh(core_axis_name="c", subcore_axis_name="s")
    scratch = [
        pltpu.VMEM((N_NODES,), jnp.float32),
        pltpu.VMEM((_CH,), jnp.float32),
    ]
    scratch += [pltpu.VMEM((_CH,), jnp.int32) for _ in range(_NBUF)]
    scratch += [pltpu.VMEM((_CH,), jnp.int32) for _ in range(_NBUF)]
    scratch += [pltpu.VMEM((_CH,), jnp.float32) for _ in range(_NBUF)]
    scratch += [pltpu.VMEM((_CH, CONV_H), jnp.float32) for _ in range(_NBUF)]
    scratch += [pltpu.SemaphoreType.DMA for _ in range(3 * _NBUF)]
    scratch += [pltpu.VMEM_SHARED((N_NODES, CONV_H), jnp.float32)]
    return pl.kernel(
        _msg_body,
        out_type=jax.ShapeDtypeStruct((_NC, N_NODES, CONV_H), jnp.float32),
        mesh=mesh,
        compiler_params=pltpu.CompilerParams(needs_layout_passes=False,
                                             use_tc_tiling_on_sc=False),
        scratch_types=scratch,
    )(h, row, col, ew, dinv, zeros)


def _tc2_body(part_ref, h_ref, dinv_ref, b1_ref, batch_ref,
              l1wt_ref, l1b_ref, l2wt_ref, l2b_ref, out_ref, pool_acc):
    i = pl.program_id(0)
    d = dinv_ref[...]
    o = part_ref[0] + part_ref[1] + h_ref[...] * (d * d) + b1_ref[...]
    o = jnp.maximum(o, 0.0)
    bid = batch_ref[...]

    @pl.when(i == 0)
    def _():
        pool_acc[...] = jnp.full((NUM_GRAPHS, CONV_H), -jnp.inf,
                                 dtype=jnp.float32)

    for g in range(NUM_GRAPHS):
        og = jnp.where(bid == g, o, -jnp.inf)
        cg = jnp.max(og, axis=0, keepdims=True)
        sl = pl.ds(g, 1)
        pool_acc[sl, :] = jnp.maximum(pool_acc[sl, :], cg)

    @pl.when(i == pl.num_programs(0) - 1)
    def _():
        p = pool_acc[...]
        z = jnp.dot(p, l1wt_ref[...], preferred_element_type=jnp.float32)
        z = jnp.maximum(z + l1b_ref[...], 0.0)
        out_ref[...] = (jnp.dot(z, l2wt_ref[...],
                                preferred_element_type=jnp.float32)
                        + l2b_ref[...])


def _tc2(part, h, dinv_col, b1, batch_col, l1wt, l1b, l2wt, l2b):
    nb = 10
    blk = N_NODES // nb
    return pl.pallas_call(
        _tc2_body,
        grid=(nb,),
        in_specs=[
            pl.BlockSpec((_NC, blk, CONV_H), lambda i: (0, i, 0)),
            pl.BlockSpec((blk, CONV_H), lambda i: (i, 0)),
            pl.BlockSpec((blk, 1), lambda i: (i, 0)),
            pl.BlockSpec((1, CONV_H), lambda i: (0, 0)),
            pl.BlockSpec((blk, 1), lambda i: (i, 0)),
            pl.BlockSpec((CONV_H, LIN_H), lambda i: (0, 0)),
            pl.BlockSpec((1, LIN_H), lambda i: (0, 0)),
            pl.BlockSpec((LIN_H, NUM_CLASSES), lambda i: (0, 0)),
            pl.BlockSpec((1, NUM_CLASSES), lambda i: (0, 0)),
        ],
        out_specs=pl.BlockSpec((NUM_GRAPHS, NUM_CLASSES), lambda i: (0, 0)),
        out_shape=jax.ShapeDtypeStruct((NUM_GRAPHS, NUM_CLASSES), jnp.float32),
        scratch_shapes=[pltpu.VMEM((NUM_GRAPHS, CONV_H), jnp.float32)],
    )(part, h, dinv_col, b1, batch_col, l1wt, l1b, l2wt, l2b)


def kernel(x, edge_index, edge_weight, batch, W1, b1, lin1_w, lin1_b,
           lin2_w, lin2_b):
    row = edge_index[0]
    col = edge_index[1]
    ew = jnp.ravel(edge_weight).astype(jnp.float32)
    x = x.astype(jnp.float32)

    degp = _deg_partials(col, ew)
    h, dinv_row = _tc1(x, W1.T, degp)
    dinv = dinv_row.reshape(N_NODES)
    zeros = jnp.zeros((N_NODES, CONV_H), jnp.float32)
    part = _msg_partials(h, row.reshape(_NCHUNK, _CH),
                         col.reshape(_NCHUNK, _CH),
                         ew.reshape(_NCHUNK, _CH), dinv, zeros)
    out = _tc2(part, h, dinv.reshape(N_NODES, 1), b1.reshape(1, CONV_H),
               batch.reshape(N_NODES, 1).astype(jnp.int32),
               lin1_w.T, lin1_b.reshape(1, LIN_H),
               lin2_w.T, lin2_b.reshape(1, NUM_CLASSES))
    return out


# speedup vs baseline: 25.3487x; 1.1876x over previous
import functools

import jax
import jax.numpy as jnp
from jax import lax
from jax.experimental import pallas as pl
from jax.experimental.pallas import tpu as pltpu
from jax.experimental.pallas import tpu_sc as plsc

N_NODES = 10000
N_EDGES = 320000
D_FEAT = 128
CONV_H = 128
LIN_H = 64
NUM_CLASSES = 10
NUM_GRAPHS = 64

_NC = 2
_NS = 16
_NW = _NC * _NS
_EPT = N_EDGES // _NW
_CH = 80
_NCHUNK = N_EDGES // _CH
_CPT = _NCHUNK // _NS
_NBUF = 5
_HF = CONV_H // 2
_GRP = _CH // 16
_RPT = N_NODES // _NS


def _deg_body(col_hbm, ew_hbm, out_hbm, col_v, w_v, deg_v):
    cid = lax.axis_index("c")
    sid = lax.axis_index("s")
    wid = sid * _NC + cid
    base = wid * _EPT
    pltpu.sync_copy(col_hbm.at[pl.ds(base, _EPT)], col_v)
    pltpu.sync_copy(ew_hbm.at[pl.ds(base, _EPT)], w_v)

    def zero(i, _):
        deg_v[pl.ds(i * 16, 16)] = jnp.zeros((16,), jnp.float32)
        return 0

    lax.fori_loop(0, N_NODES // 16, zero, 0)

    def acc(i, _):
        sl = pl.ds(i * 16, 16)
        plsc.addupdate_scatter(deg_v, [col_v[sl]], w_v[sl])
        return 0

    lax.fori_loop(0, _EPT // 16, acc, 0)
    pltpu.sync_copy(deg_v, out_hbm.at[wid])


def _deg_partials(col, ew):
    mesh = plsc.VectorSubcoreMesh(core_axis_name="c", subcore_axis_name="s")
    return pl.kernel(
        _deg_body,
        out_type=jax.ShapeDtypeStruct((_NW, N_NODES), jnp.float32),
        mesh=mesh,
        compiler_params=pltpu.CompilerParams(needs_layout_passes=False),
        scratch_types=[
            pltpu.VMEM((_EPT,), jnp.int32),
            pltpu.VMEM((_EPT,), jnp.float32),
            pltpu.VMEM((N_NODES,), jnp.float32),
        ],
    )(col, ew)


def _tc1_body(x_ref, w_ref, degp_ref, h_ref, dinv_ref):
    h = jnp.dot(x_ref[...], w_ref[...], preferred_element_type=jnp.float32)
    h_ref[0] = h[:, :_HF]
    h_ref[1] = h[:, _HF:]
    deg = jnp.sum(degp_ref[...], axis=0, keepdims=True) + 1.0
    dinv_ref[...] = lax.rsqrt(deg)


def _tc1(x, w1t, degp):
    nb = 10
    blk = N_NODES // nb
    return pl.pallas_call(
        _tc1_body,
        grid=(nb,),
        in_specs=[
            pl.BlockSpec((blk, D_FEAT), lambda i: (i, 0)),
            pl.BlockSpec((D_FEAT, CONV_H), lambda i: (0, 0)),
            pl.BlockSpec((_NW, N_NODES), lambda i: (0, 0)),
        ],
        out_specs=[
            pl.BlockSpec((_NC, blk, _HF), lambda i: (0, i, 0)),
            pl.BlockSpec((1, N_NODES), lambda i: (0, 0)),
        ],
        out_shape=[
            jax.ShapeDtypeStruct((_NC, N_NODES, _HF), jnp.float32),
            jax.ShapeDtypeStruct((1, N_NODES), jnp.float32),
        ],
    )(x, w1t, degp)


def _msg_body(h_hbm, row_hbm, col_hbm, ew_hbm, dinv_hbm, zero_hbm, part_hbm,
              dinv_v, nrm_v, *rest):
    n = _NBUF
    ridx = rest[0:n]
    cidx = rest[n:2 * n]
    wch = rest[2 * n:3 * n]
    rows = rest[3 * n:4 * n]
    esem = rest[4 * n:5 * n]
    gsem = rest[5 * n:6 * n]
    ssem = rest[6 * n:7 * n]
    accum = rest[7 * n]

    cid = lax.axis_index("c")
    sid = lax.axis_index("s")
    cbase = sid * _CPT
    hsrc = h_hbm.at[cid]

    pltpu.sync_copy(dinv_hbm, dinv_v)

    @pl.when(sid == 0)
    def _():
        pltpu.sync_copy(zero_hbm, accum)

    plsc.subcore_barrier()

    def stage(c, s):
        pltpu.async_copy(row_hbm.at[c], ridx[s], esem[s])
        pltpu.async_copy(col_hbm.at[c], cidx[s], esem[s])
        pltpu.async_copy(ew_hbm.at[c], wch[s], esem[s])

    def ewait(s):
        pltpu.make_async_copy(row_hbm.at[0], ridx[s], esem[s]).wait()
        pltpu.make_async_copy(col_hbm.at[0], cidx[s], esem[s]).wait()
        pltpu.make_async_copy(ew_hbm.at[0], wch[s], esem[s]).wait()

    def gstart(s):
        pltpu.async_copy(hsrc.at[ridx[s]], rows[s], gsem[s])

    def gwait(s):
        pltpu.make_async_copy(hsrc.at[pl.ds(0, _CH)], rows[s],
                              gsem[s]).wait()

    def swait(s):
        pltpu.make_async_copy(rows[s], accum.at[pl.ds(0, _CH)],
                              ssem[s]).wait()

    def compute_scatter(s):
        for k in range(_GRP):
            sl = pl.ds(k * 16, 16)
            dr = plsc.load_gather(dinv_v, [ridx[s][sl]])
            dc = plsc.load_gather(dinv_v, [cidx[s][sl]])
            nrm_v[sl] = dr * wch[s][sl] * dc

        def scale(g, _):
            nrm16 = nrm_v[pl.ds(g * 16, 16)]
            for i16 in range(16):
                ei = g * 16 + i16
                sc = nrm16[i16]
                for f in range(_HF // 16):
                    fs = pl.ds(f * 16, 16)
                    rows[s][ei, fs] = rows[s][ei, fs] * sc
            return 0

        lax.fori_loop(0, _GRP, scale, 0)
        pltpu.async_copy(rows[s], accum.at[cidx[s]], ssem[s], add=True)

    for s in range(_NBUF - 1):
        stage(cbase + s, s)
    for s in range(_NBUF - 2):
        ewait(s)
        gstart(s)

    def outer(t, _):
        for b in range(n):
            j = t * n + b
            gwait(b)
            compute_scatter(b)
            s4 = (b + 4) % n

            @pl.when(j + 4 < _CPT)
            def _():
                @pl.when(j >= 1)
                def _():
                    swait(s4)

                stage(cbase + j + 4, s4)

            s3 = (b + 3) % n

            @pl.when(j + 3 < _CPT)
            def _():
                ewait(s3)
                gstart(s3)

        return 0

    lax.fori_loop(0, _CPT // n, outer, 0)

    for b in range(n):
        swait(b)

    plsc.subcore_barrier()

    rb = sid * _RPT
    pltpu.sync_copy(accum.at[pl.ds(rb, _RPT)],
                    part_hbm.at[cid].at[pl.ds(rb, _RPT)])


def _msg_partials(h, row, col, ew, dinv, zeros):
    mesh = plsc.VectorSubcoreMesh(core_axis_name="c", subcore_axis_name="s")
    scratch = [
        pltpu.VMEM((N_NODES,), jnp.float32),
        pltpu.VMEM((_CH,), jnp.float32),
    ]
    scratch += [pltpu.VMEM((_CH,), jnp.int32) for _ in range(_NBUF)]
    scratch += [pltpu.VMEM((_CH,), jnp.int32) for _ in range(_NBUF)]
    scratch += [pltpu.VMEM((_CH,), jnp.float32) for _ in range(_NBUF)]
    scratch += [pltpu.VMEM((_CH, _HF), jnp.float32) for _ in range(_NBUF)]
    scratch += [pltpu.SemaphoreType.DMA for _ in range(3 * _NBUF)]
    scratch += [pltpu.VMEM_SHARED((N_NODES, _HF), jnp.float32)]
    return pl.kernel(
        _msg_body,
        out_type=jax.ShapeDtypeStruct((_NC, N_NODES, _HF), jnp.float32),
        mesh=mesh,
        compiler_params=pltpu.CompilerParams(needs_layout_passes=False,
                                             use_tc_tiling_on_sc=False),
        scratch_types=scratch,
    )(h, row, col, ew, dinv, zeros)


def _tc2_body(part_ref, h_ref, dinv_ref, b1_ref, batch_ref,
              l1wt_ref, l1b_ref, l2wt_ref, l2b_ref, out_ref, pool_acc):
    i = pl.program_id(0)
    d = dinv_ref[...]
    p = jnp.concatenate([part_ref[0], part_ref[1]], axis=-1)
    h = jnp.concatenate([h_ref[0], h_ref[1]], axis=-1)
    o = p + h * (d * d) + b1_ref[...]
    o = jnp.maximum(o, 0.0)
    bid = batch_ref[...]

    @pl.when(i == 0)
    def _():
        pool_acc[...] = jnp.full((NUM_GRAPHS, CONV_H), -jnp.inf,
                                 dtype=jnp.float32)

    for g in range(NUM_GRAPHS):
        og = jnp.where(bid == g, o, -jnp.inf)
        cg = jnp.max(og, axis=0, keepdims=True)
        sl = pl.ds(g, 1)
        pool_acc[sl, :] = jnp.maximum(pool_acc[sl, :], cg)

    @pl.when(i == pl.num_programs(0) - 1)
    def _():
        p = pool_acc[...]
        z = jnp.dot(p, l1wt_ref[...], preferred_element_type=jnp.float32)
        z = jnp.maximum(z + l1b_ref[...], 0.0)
        out_ref[...] = (jnp.dot(z, l2wt_ref[...],
                                preferred_element_type=jnp.float32)
                        + l2b_ref[...])


def _tc2(part, h, dinv_col, b1, batch_col, l1wt, l1b, l2wt, l2b):
    nb = 10
    blk = N_NODES // nb
    return pl.pallas_call(
        _tc2_body,
        grid=(nb,),
        in_specs=[
            pl.BlockSpec((_NC, blk, _HF), lambda i: (0, i, 0)),
            pl.BlockSpec((_NC, blk, _HF), lambda i: (0, i, 0)),
            pl.BlockSpec((blk, 1), lambda i: (i, 0)),
            pl.BlockSpec((1, CONV_H), lambda i: (0, 0)),
            pl.BlockSpec((blk, 1), lambda i: (i, 0)),
            pl.BlockSpec((CONV_H, LIN_H), lambda i: (0, 0)),
            pl.BlockSpec((1, LIN_H), lambda i: (0, 0)),
            pl.BlockSpec((LIN_H, NUM_CLASSES), lambda i: (0, 0)),
            pl.BlockSpec((1, NUM_CLASSES), lambda i: (0, 0)),
        ],
        out_specs=pl.BlockSpec((NUM_GRAPHS, NUM_CLASSES), lambda i: (0, 0)),
        out_shape=jax.ShapeDtypeStruct((NUM_GRAPHS, NUM_CLASSES), jnp.float32),
        scratch_shapes=[pltpu.VMEM((NUM_GRAPHS, CONV_H), jnp.float32)],
    )(part, h, dinv_col, b1, batch_col, l1wt, l1b, l2wt, l2b)


def kernel(x, edge_index, edge_weight, batch, W1, b1, lin1_w, lin1_b,
           lin2_w, lin2_b):
    row = edge_index[0]
    col = edge_index[1]
    ew = jnp.ravel(edge_weight).astype(jnp.float32)
    x = x.astype(jnp.float32)

    degp = _deg_partials(col, ew)
    h2, dinv_row = _tc1(x, W1.T, degp)
    dinv = dinv_row.reshape(N_NODES)
    zeros = jnp.zeros((N_NODES, _HF), jnp.float32)
    part = _msg_partials(h2, row.reshape(_NCHUNK, _CH),
                         col.reshape(_NCHUNK, _CH),
                         ew.reshape(_NCHUNK, _CH), dinv, zeros)
    out = _tc2(part, h2, dinv.reshape(N_NODES, 1), b1.reshape(1, CONV_H),
               batch.reshape(N_NODES, 1).astype(jnp.int32),
               lin1_w.T, lin1_b.reshape(1, LIN_H),
               lin2_w.T, lin2_b.reshape(1, NUM_CLASSES))
    return out
